# Initial kernel scaffold; baseline (speedup 1.0000x reference)
import jax
import jax.numpy as jnp
from jax.experimental import pallas as pl


def kernel(x, conv_w, conv_b, bn_gamma, bn_beta, alpha, beta):
    raise NotImplementedError("write your pallas kernel here")


# speedup vs baseline: 35.8415x; 35.8415x over previous
import numpy as np
import jax
import jax.numpy as jnp
from jax import lax
from jax.experimental import pallas as pl
from jax.experimental.pallas import tpu as pltpu
from jax.experimental.pallas import tpu_sc as plsc

B = 32
N = 4096
S = 1024
F = 64
L = 16
NCHUNK = N // L

_INT_MAX = np.int32(2147483647)
_U = 4


def _fps_body(xs_hbm, ys_hbm, zs_hbm, gx_hbm, gy_hbm, gz_hbm,
              xs, ys, zs, dist, gx, gy, gz, idxs):
    b = lax.axis_index("s") * 2 + lax.axis_index("c")
    pltpu.sync_copy(xs_hbm.at[b], xs)
    pltpu.sync_copy(ys_hbm.at[b], ys)
    pltpu.sync_copy(zs_hbm.at[b], zs)

    big = jnp.full((L,), 1e10, jnp.float32)

    @plsc.parallel_loop(0, NCHUNK)
    def _init(j):
        dist[pl.ds(j * L, L)] = big

    lane_iota = lax.iota(jnp.int32, L)

    def step(i, farthest):
        fv = jnp.full((L,), farthest, jnp.int32)
        plsc.store_scatter(idxs, [jnp.full((L,), i, jnp.int32)], fv)
        cxv = plsc.load_gather(xs, [fv])
        cyv = plsc.load_gather(ys, [fv])
        czv = plsc.load_gather(zs, [fv])

        def chunk(j, carry):
            bvs, bis = carry
            nbv, nbi = [], []
            for k in range(_U):
                jj = j * _U + k
                sl = pl.ds(jj * L, L)
                dx = xs[sl] - cxv
                dy = ys[sl] - cyv
                dz = zs[sl] - czv
                d = dx * dx + dy * dy + dz * dz
                dn = jnp.minimum(dist[sl], d)
                dist[sl] = dn
                idxv = lane_iota + jj * L
                better = dn > bvs[k]
                nbv.append(jnp.where(better, dn, bvs[k]))
                nbi.append(jnp.where(better, idxv, bis[k]))
            return tuple(nbv), tuple(nbi)

        bv0 = jnp.full((L,), -1.0, jnp.float32)
        bi0 = jnp.zeros((L,), jnp.int32)
        bvs, bis = plsc.parallel_loop(
            0, NCHUNK // _U, unroll=4,
            carry=((bv0,) * _U, (bi0,) * _U))(chunk)
        bv, bi = bvs[0], bis[0]
        for k in range(1, _U):
            take = (bvs[k] > bv) | ((bvs[k] == bv) & (bis[k] < bi))
            bv = jnp.where(take, bvs[k], bv)
            bi = jnp.where(take, bis[k], bi)
        m = jnp.max(bv)
        cand = jnp.where(bv == m, bi, _INT_MAX)
        return jnp.min(cand)

    lax.fori_loop(0, S, step, jnp.int32(0))

    @plsc.parallel_loop(0, S // L)
    def _gout(j):
        sl = pl.ds(j * L, L)
        iv = idxs[sl]
        gx[sl] = plsc.load_gather(xs, [iv])
        gy[sl] = plsc.load_gather(ys, [iv])
        gz[sl] = plsc.load_gather(zs, [iv])

    pltpu.sync_copy(gx, gx_hbm.at[b])
    pltpu.sync_copy(gy, gy_hbm.at[b])
    pltpu.sync_copy(gz, gz_hbm.at[b])


_fps = pl.kernel(
    _fps_body,
    out_type=[jax.ShapeDtypeStruct((B, S), jnp.float32)] * 3,
    mesh=plsc.VectorSubcoreMesh(core_axis_name="c", subcore_axis_name="s",
                                num_cores=2, num_subcores=16),
    scratch_types=[pltpu.VMEM((N,), jnp.float32)] * 4
                  + [pltpu.VMEM((S,), jnp.float32)] * 3
                  + [pltpu.VMEM((S,), jnp.int32)],
    compiler_params=pltpu.CompilerParams(needs_layout_passes=False),
)


def _embed_body(gx_ref, gy_ref, gz_ref, w0_ref, w1_ref, w2_ref, cb_ref,
                gamma_ref, bbeta_ref, alpha_ref, beta_ref, out_ref):
    gx = gx_ref[...]
    gy = gy_ref[...]
    gz = gz_ref[...]
    w0 = w0_ref[...]
    w1 = w1_ref[...]
    w2 = w2_ref[...]
    emb = gx * w0 + gy * w1 + gz * w2 + cb_ref[...]
    cnt = jnp.float32(B * S)
    mean_c = jnp.sum(jnp.sum(emb, axis=2, keepdims=True), axis=0,
                     keepdims=True) / cnt
    dev = emb - mean_c
    var_c = jnp.sum(jnp.sum(dev * dev, axis=2, keepdims=True), axis=0,
                    keepdims=True) / cnt
    bn = dev / jnp.sqrt(var_c + 1e-5) * gamma_ref[...] + bbeta_ref[...]
    act = jnp.where(bn > 0, bn, 0.01 * bn)
    m = jnp.sum(act, axis=2, keepdims=True) / jnp.float32(S)
    d2 = act - m
    sd = jnp.sqrt(jnp.sum(d2 * d2, axis=2, keepdims=True) / jnp.float32(S - 1))
    res = alpha_ref[...] * (d2 / (sd + 1e-5)) + beta_ref[...]
    out_ref[...] = jnp.swapaxes(res, 1, 2)


_embed = pl.pallas_call(
    _embed_body,
    out_shape=jax.ShapeDtypeStruct((B, S, F), jnp.float32),
)


def kernel(x, conv_w, conv_b, bn_gamma, bn_beta, alpha, beta):
    xs = x[:, 0, :]
    ys = x[:, 1, :]
    zs = x[:, 2, :]
    gx, gy, gz = _fps(xs, ys, zs)
    col = lambda v: v.reshape(1, F, 1)
    return _embed(gx[:, None, :], gy[:, None, :], gz[:, None, :],
                  col(conv_w[:, 0]), col(conv_w[:, 1]), col(conv_w[:, 2]),
                  col(conv_b), col(bn_gamma), col(bn_beta),
                  col(alpha[0, 0]), col(beta[0, 0]))


# speedup vs baseline: 35.9336x; 1.0026x over previous
import numpy as np
import jax
import jax.numpy as jnp
from jax import lax
from jax.experimental import pallas as pl
from jax.experimental.pallas import tpu as pltpu
from jax.experimental.pallas import tpu_sc as plsc

B = 32
N = 4096
S = 1024
F = 64
L = 16
NCHUNK = N // L

_INT_MAX = np.int32(2147483647)
_U = 2


def _fps_body(xs_hbm, ys_hbm, zs_hbm, gx_hbm, gy_hbm, gz_hbm,
              xs, ys, zs, dist, gx, gy, gz, idxs):
    b = lax.axis_index("s") * 2 + lax.axis_index("c")
    pltpu.sync_copy(xs_hbm.at[b], xs)
    pltpu.sync_copy(ys_hbm.at[b], ys)
    pltpu.sync_copy(zs_hbm.at[b], zs)

    big = jnp.full((L,), 1e10, jnp.float32)

    @plsc.parallel_loop(0, NCHUNK)
    def _init(j):
        dist[pl.ds(j * L, L)] = big

    lane_iota = lax.iota(jnp.int32, L)

    def step(i, farthest):
        fv = jnp.full((L,), farthest, jnp.int32)
        plsc.store_scatter(idxs, [jnp.full((L,), i, jnp.int32)], fv)
        cxv = plsc.load_gather(xs, [fv])
        cyv = plsc.load_gather(ys, [fv])
        czv = plsc.load_gather(zs, [fv])

        def chunk(j, carry):
            bvs, bis = carry
            nbv, nbi = [], []
            for k in range(_U):
                jj = j * _U + k
                sl = pl.ds(jj * L, L)
                dx = xs[sl] - cxv
                dy = ys[sl] - cyv
                dz = zs[sl] - czv
                d = dx * dx + dy * dy + dz * dz
                dn = jnp.minimum(dist[sl], d)
                dist[sl] = dn
                idxv = lane_iota + jj * L
                better = dn > bvs[k]
                nbv.append(jnp.where(better, dn, bvs[k]))
                nbi.append(jnp.where(better, idxv, bis[k]))
            return tuple(nbv), tuple(nbi)

        bv0 = jnp.full((L,), -1.0, jnp.float32)
        bi0 = jnp.zeros((L,), jnp.int32)
        bvs, bis = plsc.parallel_loop(
            0, NCHUNK // _U, unroll=4,
            carry=((bv0,) * _U, (bi0,) * _U))(chunk)
        bv, bi = bvs[0], bis[0]
        for k in range(1, _U):
            take = (bvs[k] > bv) | ((bvs[k] == bv) & (bis[k] < bi))
            bv = jnp.where(take, bvs[k], bv)
            bi = jnp.where(take, bis[k], bi)
        m = jnp.max(bv)
        cand = jnp.where(bv == m, bi, _INT_MAX)
        return jnp.min(cand)

    lax.fori_loop(0, S, step, jnp.int32(0))

    @plsc.parallel_loop(0, S // L)
    def _gout(j):
        sl = pl.ds(j * L, L)
        iv = idxs[sl]
        gx[sl] = plsc.load_gather(xs, [iv])
        gy[sl] = plsc.load_gather(ys, [iv])
        gz[sl] = plsc.load_gather(zs, [iv])

    pltpu.sync_copy(gx, gx_hbm.at[b])
    pltpu.sync_copy(gy, gy_hbm.at[b])
    pltpu.sync_copy(gz, gz_hbm.at[b])


_fps = pl.kernel(
    _fps_body,
    out_type=[jax.ShapeDtypeStruct((B, S), jnp.float32)] * 3,
    mesh=plsc.VectorSubcoreMesh(core_axis_name="c", subcore_axis_name="s",
                                num_cores=2, num_subcores=16),
    scratch_types=[pltpu.VMEM((N,), jnp.float32)] * 4
                  + [pltpu.VMEM((S,), jnp.float32)] * 3
                  + [pltpu.VMEM((S,), jnp.int32)],
    compiler_params=pltpu.CompilerParams(needs_layout_passes=False),
)


def _embed_body(gx_ref, gy_ref, gz_ref, w0_ref, w1_ref, w2_ref, cb_ref,
                gamma_ref, bbeta_ref, alpha_ref, beta_ref, out_ref):
    gx = gx_ref[...]
    gy = gy_ref[...]
    gz = gz_ref[...]
    w0 = w0_ref[...]
    w1 = w1_ref[...]
    w2 = w2_ref[...]
    emb = gx * w0 + gy * w1 + gz * w2 + cb_ref[...]
    cnt = jnp.float32(B * S)
    mean_c = jnp.sum(jnp.sum(emb, axis=2, keepdims=True), axis=0,
                     keepdims=True) / cnt
    dev = emb - mean_c
    var_c = jnp.sum(jnp.sum(dev * dev, axis=2, keepdims=True), axis=0,
                    keepdims=True) / cnt
    bn = dev / jnp.sqrt(var_c + 1e-5) * gamma_ref[...] + bbeta_ref[...]
    act = jnp.where(bn > 0, bn, 0.01 * bn)
    m = jnp.sum(act, axis=2, keepdims=True) / jnp.float32(S)
    d2 = act - m
    sd = jnp.sqrt(jnp.sum(d2 * d2, axis=2, keepdims=True) / jnp.float32(S - 1))
    res = alpha_ref[...] * (d2 / (sd + 1e-5)) + beta_ref[...]
    out_ref[...] = jnp.swapaxes(res, 1, 2)


_embed = pl.pallas_call(
    _embed_body,
    out_shape=jax.ShapeDtypeStruct((B, S, F), jnp.float32),
)


def kernel(x, conv_w, conv_b, bn_gamma, bn_beta, alpha, beta):
    xs = x[:, 0, :]
    ys = x[:, 1, :]
    zs = x[:, 2, :]
    gx, gy, gz = _fps(xs, ys, zs)
    col = lambda v: v.reshape(1, F, 1)
    return _embed(gx[:, None, :], gy[:, None, :], gz[:, None, :],
                  col(conv_w[:, 0]), col(conv_w[:, 1]), col(conv_w[:, 2]),
                  col(conv_b), col(bn_gamma), col(bn_beta),
                  col(alpha[0, 0]), col(beta[0, 0]))
